# baseline (speedup 1.0000x reference)
import functools

import jax
import jax.numpy as jnp
from jax import lax
from jax.experimental import pallas as pl
from jax.experimental.pallas import tpu as pltpu
from jax.experimental.pallas import tpu_sc as plsc

VOCAB = 1000000
EMB = 128
ENC = 128
HID = 128
B = 1024
L = 50
N_ROWS = B * L


_CH = 80


def _sc_gather(table, x, n_ch, nw, num_cores):
    mesh = plsc.VectorSubcoreMesh(core_axis_name="c", subcore_axis_name="s")
    b_per_w = n_ch * _CH

    @functools.partial(
        pl.kernel,
        out_type=jax.ShapeDtypeStruct((N_ROWS, EMB), jnp.float32),
        mesh=mesh,
        scratch_types=[
            pltpu.VMEM((n_ch, _CH), jnp.int32),
            pltpu.VMEM((n_ch, _CH), jnp.int32),
            pltpu.VMEM((_CH, EMB), jnp.float32),
            pltpu.VMEM((_CH, EMB), jnp.float32),
            pltpu.SemaphoreType.DMA,
            pltpu.SemaphoreType.DMA,
            pltpu.SemaphoreType.DMA,
            pltpu.SemaphoreType.DMA,
            pltpu.SemaphoreType.DMA,
        ],
    )
    def k(table_hbm, x_hbm, out_hbm, addr_v, idx_v, rows0, rows1,
          asem, g0, g1, o0, o1):
        wid = lax.axis_index("s") * num_cores + lax.axis_index("c")
        base = wid * b_per_w
        lanes = lax.iota(jnp.int32, 16)
        for i in range(b_per_w // 16):
            kv = (base + i * 16) + lanes
            lpos = jax.lax.shift_right_logical(kv, 10)
            bpos = jax.lax.bitwise_and(kv, B - 1)
            addr_v[(i * 16) // _CH, pl.ds((i * 16) % _CH, 16)] = \
                bpos * L + lpos
        ah = [pltpu.async_copy(x_hbm.at[addr_v.at[j]], idx_v.at[j], asem)
              for j in range(n_ch)]
        rows = (rows0, rows1)
        gsem = (g0, g1)
        osem = (o0, o1)
        gh = [None, None]
        oh = [None, None]
        for j in range(n_ch + 1):
            s = j % 2
            if j < n_ch:
                if oh[s] is not None:
                    oh[s].wait()
                    oh[s] = None
                ah[j].wait()
                gh[s] = pltpu.async_copy(
                    table_hbm.at[idx_v.at[j]], rows[s], gsem[s]
                )
            if j >= 1:
                p = (j - 1) % 2
                gh[p].wait()
                oh[p] = pltpu.async_copy(
                    rows[p], out_hbm.at[pl.ds(base + (j - 1) * _CH, _CH)], osem[p]
                )
        for p in range(2):
            if oh[p] is not None:
                oh[p].wait()

    return k(table, x)


def _rnn_kernel(emb_ref, wc_ref, bc_ref, wif_ref, bif_ref, whf_ref, bhf_ref,
                wib_ref, bib_ref, whb_ref, bhb_ref, out_ref,
                h_ref, m_ref):
    f32 = jnp.float32
    bf16 = jnp.bfloat16
    G = 3 * HID
    wc = wc_ref[...]
    a_f = lax.dot_general(wc, wif_ref[...], (((0,), (1,)), ((), ())),
                          preferred_element_type=f32)
    a_b = lax.dot_general(wc, wib_ref[...], (((0,), (1,)), ((), ())),
                          preferred_element_type=f32)
    c_f = lax.dot_general(bc_ref[...], wif_ref[...], (((1,), (1,)), ((), ())),
                          preferred_element_type=f32) + bif_ref[...]
    c_b = lax.dot_general(bc_ref[...], wib_ref[...], (((1,), (1,)), ((), ())),
                          preferred_element_type=f32) + bib_ref[...]
    a_f16 = a_f.astype(bf16)
    a_b16 = a_b.astype(bf16)
    half = jnp.bfloat16(0.5)
    zgh = jnp.zeros((G, HID), f32)
    wh_cat = jnp.concatenate([
        jnp.concatenate([whf_ref[...], zgh], axis=1),
        jnp.concatenate([zgh, whb_ref[...]], axis=1),
    ], axis=0).astype(bf16)
    bhf = bhf_ref[...]
    bhb = bhb_ref[...]
    brz_f = (c_f[:, :2 * HID] + bhf[:, :2 * HID]).astype(bf16)
    brz_b = (c_b[:, :2 * HID] + bhb[:, :2 * HID]).astype(bf16)
    cn_f = c_f[:, 2 * HID:].astype(bf16)
    cn_b = c_b[:, 2 * HID:].astype(bf16)
    bhn_f = bhf[:, 2 * HID:].astype(bf16)
    bhn_b = bhb[:, 2 * HID:].astype(bf16)

    h_ref[...] = jnp.zeros((B, 2 * HID), bf16)
    m_ref[...] = jnp.full((B, 2 * HID), -jnp.inf, bf16)

    def gates(gi_d, gh_d, h_d, brz, cn, bhn):
        s = gi_d[:, :2 * HID] + gh_d[:, :2 * HID] + brz
        r = half * jnp.tanh(half * s[:, :HID]) + half
        z = half * jnp.tanh(half * s[:, HID:]) + half
        n = jnp.tanh((gi_d[:, 2 * HID:] + cn) + r * (gh_d[:, 2 * HID:] + bhn))
        return n + z * (h_d - n)

    def step(t, _):
        gi_f = lax.dot_general(emb_ref[t].astype(bf16), a_f16,
                               (((1,), (0,)), ((), ())),
                               preferred_element_type=f32).astype(bf16)
        gi_b = lax.dot_general(emb_ref[L - 1 - t].astype(bf16), a_b16,
                               (((1,), (0,)), ((), ())),
                               preferred_element_type=f32).astype(bf16)
        hc = h_ref[...]
        gh = lax.dot_general(hc, wh_cat,
                             (((1,), (1,)), ((), ())),
                             preferred_element_type=f32).astype(bf16)
        h_f = gates(gi_f, gh[:, :G], hc[:, :HID], brz_f, cn_f, bhn_f)
        h_b = gates(gi_b, gh[:, G:], hc[:, HID:], brz_b, cn_b, bhn_b)
        h_ref[:, :HID] = h_f
        h_ref[:, HID:] = h_b
        m_ref[:, :HID] = jnp.maximum(m_ref[:, :HID], h_f)
        m_ref[:, HID:] = jnp.maximum(m_ref[:, HID:], h_b)
        return 0

    lax.fori_loop(0, L, step, 0)
    out_ref[...] = m_ref[...].astype(f32)


def _tc_rnn(emb, wc, bc, wif, bif, whf, bhf, wib, bib, whb, bhb):
    return pl.pallas_call(
        _rnn_kernel,
        out_shape=jax.ShapeDtypeStruct((B, 2 * HID), jnp.float32),
        scratch_shapes=[
            pltpu.VMEM((B, 2 * HID), jnp.bfloat16),
            pltpu.VMEM((B, 2 * HID), jnp.bfloat16),
        ],
    )(emb, wc, bc, wif, bif, whf, bhf, wib, bib, whb, bhb)


def kernel(x, table, W_c, b_c, W_ih_f, W_hh_f, b_ih_f, b_hh_f,
           W_ih_b, W_hh_b, b_ih_b, b_hh_b):
    info = plsc.get_sparse_core_info()
    nw = info.num_cores * info.num_subcores
    n_ch = N_ROWS // (nw * _CH)
    emb = _sc_gather(table, x.reshape(-1), n_ch, nw, info.num_cores)
    emb = emb.reshape(L, B, EMB)
    return _tc_rnn(
        emb, W_c, b_c.reshape(1, ENC),
        W_ih_f, b_ih_f.reshape(1, 3 * HID), W_hh_f, b_hh_f.reshape(1, 3 * HID),
        W_ih_b, b_ih_b.reshape(1, 3 * HID), W_hh_b, b_hh_b.reshape(1, 3 * HID),
    )


# speedup vs baseline: 1.0682x; 1.0682x over previous
import functools

import jax
import jax.numpy as jnp
from jax import lax
from jax.experimental import pallas as pl
from jax.experimental.pallas import tpu as pltpu
from jax.experimental.pallas import tpu_sc as plsc

VOCAB = 1000000
EMB = 128
ENC = 128
HID = 128
B = 1024
L = 50
N_ROWS = B * L


_CH = 80


def _sc_gather(table, x, n_ch, nw, num_cores):
    mesh = plsc.VectorSubcoreMesh(core_axis_name="c", subcore_axis_name="s")
    b_per_w = n_ch * _CH

    @functools.partial(
        pl.kernel,
        out_type=jax.ShapeDtypeStruct((N_ROWS, EMB), jnp.float32),
        mesh=mesh,
        scratch_types=[
            pltpu.VMEM((n_ch, _CH), jnp.int32),
            pltpu.VMEM((n_ch, _CH), jnp.int32),
            pltpu.VMEM((_CH, EMB), jnp.float32),
            pltpu.VMEM((_CH, EMB), jnp.float32),
            pltpu.SemaphoreType.DMA,
            pltpu.SemaphoreType.DMA,
            pltpu.SemaphoreType.DMA,
            pltpu.SemaphoreType.DMA,
            pltpu.SemaphoreType.DMA,
        ],
    )
    def k(table_hbm, x_hbm, out_hbm, addr_v, idx_v, rows0, rows1,
          asem, g0, g1, o0, o1):
        wid = lax.axis_index("s") * num_cores + lax.axis_index("c")
        base = wid * b_per_w
        lanes = lax.iota(jnp.int32, 16)
        for i in range(b_per_w // 16):
            kv = (base + i * 16) + lanes
            lpos = jax.lax.shift_right_logical(kv, 10)
            bpos = jax.lax.bitwise_and(kv, B - 1)
            addr_v[(i * 16) // _CH, pl.ds((i * 16) % _CH, 16)] = \
                bpos * L + lpos
        ah = [pltpu.async_copy(x_hbm.at[addr_v.at[j]], idx_v.at[j], asem)
              for j in range(n_ch)]
        rows = (rows0, rows1)
        gsem = (g0, g1)
        osem = (o0, o1)
        gh = [None, None]
        oh = [None, None]
        for j in range(n_ch + 1):
            s = j % 2
            if j < n_ch:
                if oh[s] is not None:
                    oh[s].wait()
                    oh[s] = None
                ah[j].wait()
                gh[s] = pltpu.async_copy(
                    table_hbm.at[idx_v.at[j]], rows[s], gsem[s]
                )
            if j >= 1:
                p = (j - 1) % 2
                gh[p].wait()
                oh[p] = pltpu.async_copy(
                    rows[p], out_hbm.at[pl.ds(base + (j - 1) * _CH, _CH)], osem[p]
                )
        for p in range(2):
            if oh[p] is not None:
                oh[p].wait()

    return k(table, x)


def _rnn_kernel(emb_ref, wc_ref, bc_ref, wif_ref, bif_ref, whf_ref, bhf_ref,
                wib_ref, bib_ref, whb_ref, bhb_ref, out_ref,
                h_ref, m_ref):
    f32 = jnp.float32
    bf16 = jnp.bfloat16
    G = 3 * HID
    wc = wc_ref[...]
    a_f = lax.dot_general(wc, wif_ref[...], (((0,), (1,)), ((), ())),
                          preferred_element_type=f32)
    a_b = lax.dot_general(wc, wib_ref[...], (((0,), (1,)), ((), ())),
                          preferred_element_type=f32)
    c_f = lax.dot_general(bc_ref[...], wif_ref[...], (((1,), (1,)), ((), ())),
                          preferred_element_type=f32) + bif_ref[...]
    c_b = lax.dot_general(bc_ref[...], wib_ref[...], (((1,), (1,)), ((), ())),
                          preferred_element_type=f32) + bib_ref[...]
    whf_t = lax.transpose(whf_ref[...], (1, 0))
    whb_t = lax.transpose(whb_ref[...], (1, 0))
    bhf = bhf_ref[...]
    bhb = bhb_ref[...]
    brz_f = c_f[:, :2 * HID] + bhf[:, :2 * HID]
    brz_b = c_b[:, :2 * HID] + bhb[:, :2 * HID]
    cn_f = c_f[:, 2 * HID:]
    cn_b = c_b[:, 2 * HID:]
    bhn_f = bhf[:, 2 * HID:]
    bhn_b = bhb[:, 2 * HID:]

    h_ref[...] = jnp.zeros((B, 2 * HID), f32)
    m_ref[...] = jnp.full((B, 2 * HID), -jnp.inf, f32)

    def gates(gi_d, gh_d, h_d, brz, cn, bhn):
        s = gi_d[:, :2 * HID] + gh_d[:, :2 * HID] + brz
        r = 0.5 * jnp.tanh(0.5 * s[:, :HID]) + 0.5
        z = 0.5 * jnp.tanh(0.5 * s[:, HID:]) + 0.5
        n = jnp.tanh((gi_d[:, 2 * HID:] + cn) + r * (gh_d[:, 2 * HID:] + bhn))
        return n + z * (h_d - n)

    def substep(t):
        gi_f = lax.dot_general(emb_ref[t], a_f, (((1,), (0,)), ((), ())),
                               preferred_element_type=f32)
        gi_b = lax.dot_general(emb_ref[L - 1 - t], a_b,
                               (((1,), (0,)), ((), ())),
                               preferred_element_type=f32)
        hc = h_ref[...]
        gh_f = lax.dot_general(hc[:, :HID], whf_t, (((1,), (0,)), ((), ())),
                               preferred_element_type=f32)
        gh_b = lax.dot_general(hc[:, HID:], whb_t, (((1,), (0,)), ((), ())),
                               preferred_element_type=f32)
        h_f = gates(gi_f, gh_f, hc[:, :HID], brz_f, cn_f, bhn_f)
        h_b = gates(gi_b, gh_b, hc[:, HID:], brz_b, cn_b, bhn_b)
        h_ref[:, :HID] = h_f
        h_ref[:, HID:] = h_b
        m_ref[:, :HID] = jnp.maximum(m_ref[:, :HID], h_f)
        m_ref[:, HID:] = jnp.maximum(m_ref[:, HID:], h_b)

    def step(t, _):
        substep(2 * t)
        substep(2 * t + 1)
        return 0

    lax.fori_loop(0, L // 2, step, 0)
    out_ref[...] = m_ref[...]


def _tc_rnn(emb, wc, bc, wif, bif, whf, bhf, wib, bib, whb, bhb):
    return pl.pallas_call(
        _rnn_kernel,
        out_shape=jax.ShapeDtypeStruct((B, 2 * HID), jnp.float32),
        scratch_shapes=[
            pltpu.VMEM((B, 2 * HID), jnp.float32),
            pltpu.VMEM((B, 2 * HID), jnp.float32),
        ],
    )(emb, wc, bc, wif, bif, whf, bhf, wib, bib, whb, bhb)


def kernel(x, table, W_c, b_c, W_ih_f, W_hh_f, b_ih_f, b_hh_f,
           W_ih_b, W_hh_b, b_ih_b, b_hh_b):
    info = plsc.get_sparse_core_info()
    nw = info.num_cores * info.num_subcores
    n_ch = N_ROWS // (nw * _CH)
    emb = _sc_gather(table, x.reshape(-1), n_ch, nw, info.num_cores)
    emb = emb.reshape(L, B, EMB)
    return _tc_rnn(
        emb, W_c, b_c.reshape(1, ENC),
        W_ih_f, b_ih_f.reshape(1, 3 * HID), W_hh_f, b_hh_f.reshape(1, 3 * HID),
        W_ih_b, b_ih_b.reshape(1, 3 * HID), W_hh_b, b_hh_b.reshape(1, 3 * HID),
    )


# speedup vs baseline: 1.1356x; 1.0631x over previous
import functools

import jax
import jax.numpy as jnp
from jax import lax
from jax.experimental import pallas as pl
from jax.experimental.pallas import tpu as pltpu
from jax.experimental.pallas import tpu_sc as plsc

VOCAB = 1000000
EMB = 128
ENC = 128
HID = 128
B = 1024
L = 50
NSPLIT = 2
BH = B // NSPLIT
BH_BITS = 9

_CH = 80


def _sc_gather(table, xflat):
    info = plsc.get_sparse_core_info()
    nw = info.num_cores * info.num_subcores
    num_cores = info.num_cores
    n_rows = BH * L
    b_per_w = n_rows // nw
    n_ch = b_per_w // _CH
    mesh = plsc.VectorSubcoreMesh(core_axis_name="c", subcore_axis_name="s")

    @functools.partial(
        pl.kernel,
        out_type=jax.ShapeDtypeStruct((n_rows, EMB), jnp.float32),
        mesh=mesh,
        scratch_types=[
            pltpu.VMEM((n_ch, _CH), jnp.int32),
            pltpu.VMEM((n_ch, _CH), jnp.int32),
            pltpu.VMEM((_CH, EMB), jnp.float32),
            pltpu.VMEM((_CH, EMB), jnp.float32),
            pltpu.SemaphoreType.DMA,
            pltpu.SemaphoreType.DMA,
            pltpu.SemaphoreType.DMA,
            pltpu.SemaphoreType.DMA,
            pltpu.SemaphoreType.DMA,
        ],
    )
    def k(table_hbm, x_hbm, out_hbm, addr_v, idx_v, rows0, rows1,
          asem, g0, g1, o0, o1):
        wid = lax.axis_index("s") * num_cores + lax.axis_index("c")
        base = wid * b_per_w
        lanes = lax.iota(jnp.int32, 16)
        for i in range(b_per_w // 16):
            kv = (base + i * 16) + lanes
            lpos = jax.lax.shift_right_logical(kv, BH_BITS)
            bpos = jax.lax.bitwise_and(kv, BH - 1)
            addr_v[(i * 16) // _CH, pl.ds((i * 16) % _CH, 16)] = \
                bpos * L + lpos
        ah = [pltpu.async_copy(x_hbm.at[addr_v.at[j]], idx_v.at[j], asem)
              for j in range(n_ch)]
        rows = (rows0, rows1)
        gsem = (g0, g1)
        osem = (o0, o1)
        gh = [None, None]
        oh = [None, None]
        for j in range(n_ch + 1):
            s = j % 2
            if j < n_ch:
                if oh[s] is not None:
                    oh[s].wait()
                    oh[s] = None
                ah[j].wait()
                gh[s] = pltpu.async_copy(
                    table_hbm.at[idx_v.at[j]], rows[s], gsem[s]
                )
            if j >= 1:
                p = (j - 1) % 2
                gh[p].wait()
                oh[p] = pltpu.async_copy(
                    rows[p], out_hbm.at[pl.ds(base + (j - 1) * _CH, _CH)], osem[p]
                )
        for p in range(2):
            if oh[p] is not None:
                oh[p].wait()

    return k(table, xflat)


def _rnn_kernel(emb_ref, wc_ref, bc_ref, wif_ref, bif_ref, whf_ref, bhf_ref,
                wib_ref, bib_ref, whb_ref, bhb_ref, out_ref,
                h_ref, m_ref):
    f32 = jnp.float32
    wc = wc_ref[...]
    a_f = lax.dot_general(wc, wif_ref[...], (((0,), (1,)), ((), ())),
                          preferred_element_type=f32)
    a_b = lax.dot_general(wc, wib_ref[...], (((0,), (1,)), ((), ())),
                          preferred_element_type=f32)
    c_f = lax.dot_general(bc_ref[...], wif_ref[...], (((1,), (1,)), ((), ())),
                          preferred_element_type=f32) + bif_ref[...]
    c_b = lax.dot_general(bc_ref[...], wib_ref[...], (((1,), (1,)), ((), ())),
                          preferred_element_type=f32) + bib_ref[...]
    whf_t = lax.transpose(whf_ref[...], (1, 0))
    whb_t = lax.transpose(whb_ref[...], (1, 0))
    bhf = bhf_ref[...]
    bhb = bhb_ref[...]
    brz_f = c_f[:, :2 * HID] + bhf[:, :2 * HID]
    brz_b = c_b[:, :2 * HID] + bhb[:, :2 * HID]
    cn_f = c_f[:, 2 * HID:]
    cn_b = c_b[:, 2 * HID:]
    bhn_f = bhf[:, 2 * HID:]
    bhn_b = bhb[:, 2 * HID:]

    h_ref[...] = jnp.zeros((BH, 2 * HID), f32)
    m_ref[...] = jnp.full((BH, 2 * HID), -jnp.inf, f32)

    def gates(gi_d, gh_d, h_d, brz, cn, bhn):
        s = gi_d[:, :2 * HID] + gh_d[:, :2 * HID] + brz
        r = 0.5 * jnp.tanh(0.5 * s[:, :HID]) + 0.5
        z = 0.5 * jnp.tanh(0.5 * s[:, HID:]) + 0.5
        n = jnp.tanh((gi_d[:, 2 * HID:] + cn) + r * (gh_d[:, 2 * HID:] + bhn))
        return n + z * (h_d - n)

    def substep(t):
        gi_f = lax.dot_general(emb_ref[t], a_f, (((1,), (0,)), ((), ())),
                               preferred_element_type=f32)
        gi_b = lax.dot_general(emb_ref[L - 1 - t], a_b,
                               (((1,), (0,)), ((), ())),
                               preferred_element_type=f32)
        hc = h_ref[...]
        gh_f = lax.dot_general(hc[:, :HID], whf_t, (((1,), (0,)), ((), ())),
                               preferred_element_type=f32)
        gh_b = lax.dot_general(hc[:, HID:], whb_t, (((1,), (0,)), ((), ())),
                               preferred_element_type=f32)
        h_f = gates(gi_f, gh_f, hc[:, :HID], brz_f, cn_f, bhn_f)
        h_b = gates(gi_b, gh_b, hc[:, HID:], brz_b, cn_b, bhn_b)
        h_ref[:, :HID] = h_f
        h_ref[:, HID:] = h_b
        m_ref[:, :HID] = jnp.maximum(m_ref[:, :HID], h_f)
        m_ref[:, HID:] = jnp.maximum(m_ref[:, HID:], h_b)

    def step(t, _):
        substep(2 * t)
        substep(2 * t + 1)
        return 0

    lax.fori_loop(0, L // 2, step, 0)
    out_ref[...] = m_ref[...]


def _tc_rnn(emb, wc, bc, wif, bif, whf, bhf, wib, bib, whb, bhb):
    return pl.pallas_call(
        _rnn_kernel,
        out_shape=jax.ShapeDtypeStruct((BH, 2 * HID), jnp.float32),
        scratch_shapes=[
            pltpu.VMEM((BH, 2 * HID), jnp.float32),
            pltpu.VMEM((BH, 2 * HID), jnp.float32),
        ],
    )(emb, wc, bc, wif, bif, whf, bhf, wib, bib, whb, bhb)


def kernel(x, table, W_c, b_c, W_ih_f, W_hh_f, b_ih_f, b_hh_f,
           W_ih_b, W_hh_b, b_ih_b, b_hh_b):
    weights = (
        W_c, b_c.reshape(1, ENC),
        W_ih_f, b_ih_f.reshape(1, 3 * HID), W_hh_f, b_hh_f.reshape(1, 3 * HID),
        W_ih_b, b_ih_b.reshape(1, 3 * HID), W_hh_b, b_hh_b.reshape(1, 3 * HID),
    )
    outs = []
    for m in range(NSPLIT):
        xflat = x[m * BH:(m + 1) * BH].reshape(-1)
        emb = _sc_gather(table, xflat).reshape(L, BH, EMB)
        outs.append(_tc_rnn(emb, *weights))
    return jnp.concatenate(outs, axis=0)
